# Initial kernel scaffold; baseline (speedup 1.0000x reference)
import jax
import jax.numpy as jnp
from jax.experimental import pallas as pl


def kernel(feat, centers, label):
    raise NotImplementedError("write your pallas kernel here")


# speedup vs baseline: 1.0832x; 1.0832x over previous
import functools

import jax
import jax.numpy as jnp
from jax import lax
from jax.experimental import pallas as pl
from jax.experimental.pallas import tpu as pltpu
from jax.experimental.pallas import tpu_sc as plsc

_B = 4096
_D = 512
_NC = 2
_NS = 16
_NW = _NC * _NS
_BPW = _B // _NW
_CH = 32
_NCHUNK = _BPW // _CH
_LANES = 16
_SCALE = 0.5 / _B

_mesh = plsc.VectorSubcoreMesh(core_axis_name="c", subcore_axis_name="s")


@functools.partial(
    pl.kernel,
    out_type=jax.ShapeDtypeStruct((_NW, _LANES), jnp.float32),
    mesh=_mesh,
    scratch_types=[
        pltpu.VMEM((_NCHUNK, _CH), jnp.int32),
        pltpu.VMEM((2, _CH, _D), jnp.float32),
        pltpu.VMEM((2, _CH, _D), jnp.float32),
        pltpu.VMEM((_LANES,), jnp.float32),
        pltpu.SemaphoreType.DMA,
        pltpu.SemaphoreType.DMA,
        pltpu.SemaphoreType.DMA,
        pltpu.SemaphoreType.DMA,
    ],
)
def _center_loss_partial(feat_hbm, centers_hbm, label_hbm, out_hbm,
                         idx_v, cbuf, fbuf, accbuf,
                         csem0, csem1, fsem0, fsem1):
    wid = lax.axis_index("s") * _NC + lax.axis_index("c")
    base = wid * _BPW

    pltpu.sync_copy(label_hbm.at[wid], idx_v)

    csems = (csem0, csem1)
    fsems = (fsem0, fsem1)

    def start(j):
        b = j % 2
        cc = pltpu.async_copy(centers_hbm.at[idx_v.at[j]], cbuf.at[b], csems[b])
        fc = pltpu.async_copy(feat_hbm.at[pl.ds(base + j * _CH, _CH)],
                              fbuf.at[b], fsems[b])
        return cc, fc

    zero = jnp.zeros((_LANES,), jnp.float32)
    accs = [zero, zero, zero, zero]

    pending = start(0)
    for j in range(_NCHUNK):
        nxt = start(j + 1) if j + 1 < _NCHUNK else None
        pending[0].wait()
        pending[1].wait()
        b = j % 2

        def row(i, acc4, _b=b):
            a = list(acc4)
            for k in range(_D // _LANES):
                d = (fbuf[_b, i, pl.ds(k * _LANES, _LANES)]
                     - cbuf[_b, i, pl.ds(k * _LANES, _LANES)])
                a[k % 4] = a[k % 4] + d * d
            return tuple(a)

        accs = list(lax.fori_loop(0, _CH, row, tuple(accs)))
        pending = nxt

    total = ((accs[0] + accs[1]) + (accs[2] + accs[3])) * _SCALE
    accbuf[...] = total
    pltpu.sync_copy(accbuf, out_hbm.at[wid])


def kernel(feat, centers, label):
    label3 = label.reshape(_NW, _NCHUNK, _CH)
    partials = _center_loss_partial(feat, centers, label3)
    return jnp.sum(partials)


# speedup vs baseline: 1.1081x; 1.0230x over previous
import functools

import jax
import jax.numpy as jnp
from jax import lax
from jax.experimental import pallas as pl
from jax.experimental.pallas import tpu as pltpu
from jax.experimental.pallas import tpu_sc as plsc

_B = 4096
_D = 512
_NC = 2
_NS = 16
_NW = _NC * _NS
_BPW = _B // _NW
_CH = 32
_NCHUNK = _BPW // _CH
_LANES = 16
_SCALE = 0.5 / _B

_mesh = plsc.VectorSubcoreMesh(core_axis_name="c", subcore_axis_name="s")


_NBUF = 3


@functools.partial(
    pl.kernel,
    out_type=jax.ShapeDtypeStruct((_NW, _LANES), jnp.float32),
    mesh=_mesh,
    scratch_types=[
        pltpu.VMEM((_BPW,), jnp.int32),
        pltpu.VMEM((_NBUF, _CH, _D), jnp.float32),
        pltpu.VMEM((_NBUF, _CH, _D), jnp.float32),
        pltpu.VMEM((_LANES,), jnp.float32),
        pltpu.SemaphoreType.DMA,
        pltpu.SemaphoreType.DMA,
        pltpu.SemaphoreType.DMA,
        pltpu.SemaphoreType.DMA,
        pltpu.SemaphoreType.DMA,
        pltpu.SemaphoreType.DMA,
    ],
)
def _center_loss_partial(feat_hbm, centers_hbm, label_hbm, out_hbm,
                         idx_v, cbuf, fbuf, accbuf,
                         csem0, csem1, csem2, fsem0, fsem1, fsem2):
    wid = lax.axis_index("s") * _NC + lax.axis_index("c")
    base = wid * _BPW

    pltpu.sync_copy(label_hbm.at[pl.ds(base, _BPW)], idx_v)

    csems = (csem0, csem1, csem2)
    fsems = (fsem0, fsem1, fsem2)

    def start(j):
        b = j % _NBUF
        cc = pltpu.async_copy(centers_hbm.at[idx_v.at[pl.ds(j * _CH, _CH)]],
                              cbuf.at[b], csems[b])
        fc = pltpu.async_copy(feat_hbm.at[pl.ds(base + j * _CH, _CH)],
                              fbuf.at[b], fsems[b])
        return cc, fc

    zero = jnp.zeros((_LANES,), jnp.float32)
    accs = (zero, zero, zero, zero)

    pending = [start(j) for j in range(_NBUF - 1)]
    for j in range(_NCHUNK):
        if j + _NBUF - 1 < _NCHUNK:
            pending.append(start(j + _NBUF - 1))
        cc, fc = pending.pop(0)
        cc.wait()
        fc.wait()
        b = j % _NBUF

        @plsc.parallel_loop(0, _CH, step=1, unroll=2, carry=accs)
        def accs(i, acc4, _b=b):
            a = list(acc4)
            for k in range(_D // _LANES):
                d = (fbuf[_b, i, pl.ds(k * _LANES, _LANES)]
                     - cbuf[_b, i, pl.ds(k * _LANES, _LANES)])
                a[k % 4] = a[k % 4] + d * d
            return tuple(a)

    total = ((accs[0] + accs[1]) + (accs[2] + accs[3])) * _SCALE
    accbuf[...] = total
    pltpu.sync_copy(accbuf, out_hbm.at[wid])


def kernel(feat, centers, label):
    partials = _center_loss_partial(feat, centers, label)
    return jnp.sum(partials)
